# baseline (speedup 1.0000x reference)
import jax
import jax.numpy as jnp
from jax import lax
from jax.experimental import pallas as pl
from jax.experimental.pallas import tpu as pltpu
from jax.experimental.pallas import tpu_sc as plsc

BATCH = 16384
N_USERS = 1000000
N_ITEMS = 100000
EMB_DIM = 32
CHUNK = 1024
U_TAIL = (N_USERS // 128) * 128
I_TAIL = (N_ITEMS // 128) * 128
U_SPAN = 31744
U_CHUNKS = 31
U_LAST = U_TAIL - CHUNK
I_SPAN = 7168
I_CHUNKS = 7
I_LAST = I_TAIL - CHUNK
CAP = 1024
GCAP = 256
BIG = 1 << 28


def _gmf_body(ut_hbm, it_hbm, u_hbm, i_hbm, w_hbm, ut_tl_hbm, it_tl_hbm,
              out_hbm,
              uidx, iidx, chk, lv, lb, go, gb, stage, gidx, utail, itail,
              wv, ovals, opos, sp_rows, sem, sem2):
    nc = lax.axis_index("c")
    ns = lax.axis_index("s")
    wid = ns * 2 + nc
    e_iota = lax.iota(jnp.int32, 16)
    zeros16 = jnp.zeros((16,), jnp.int32)

    pltpu.sync_copy(u_hbm, uidx)
    pltpu.sync_copy(i_hbm, iidx)
    pltpu.sync_copy(w_hbm, wv)
    pltpu.sync_copy(ut_tl_hbm, utail)
    pltpu.sync_copy(it_tl_hbm, itail)

    neg1 = jnp.full((16,), -1, jnp.int32)

    def clear(r, n):
        def bd(t, _):
            r[pl.ds(t * 16, 16)] = neg1
            return 0
        lax.fori_loop(0, n // 16, bd, 0, unroll=1)

    def prescan(lo, hi, vals_ref, extra_filter):
        def bd(t, cnt):
            v = vals_ref[pl.ds(t * 16, 16)]
            b = t * 16 + e_iota
            m = (v >= lo) & (v < hi) & extra_filter(t)
            plsc.store_compressed(lv.at[pl.ds(cnt, 16)], v, mask=m)
            plsc.store_compressed(lb.at[pl.ds(cnt, 16)], b, mask=m)
            return cnt + plsc.all_reduce_population_count(m)[0]
        return lax.fori_loop(0, BATCH // 16, bd, 0, unroll=1)

    def match_chunk(n, cbase, cend):
        def bd(t, cnt):
            v = lv[pl.ds(t * 16, 16)]
            b = lb[pl.ds(t * 16, 16)]
            m = (v >= cbase) & (v < cend) & (t * 16 + e_iota < n)
            plsc.store_compressed(go.at[pl.ds(cnt, 16)], v - cbase, mask=m)
            plsc.store_compressed(gb.at[pl.ds(cnt, 16)], b, mask=m)
            return cnt + plsc.all_reduce_population_count(m)[0]
        return lax.fori_loop(0, (n + 15) // 16, bd, 0, unroll=1)

    def fetch_wait(tbl, cbase):
        cbase = pl.multiple_of(cbase, 128)
        cps = [pltpu.async_copy(tbl.at[fb, :, pl.ds(cbase, CHUNK)],
                                chk.at[fb], sem) for fb in range(4)]
        for c in cps:
            c.wait()

    def sc_half(t):
        u = uidx[pl.ds(t * 16, 16)]
        return ((u // U_SPAN) & 1) == nc

    a_lo = ns * I_SPAN
    na = prescan(a_lo, a_lo + I_SPAN, iidx, sc_half)

    def a_chunk(k, _):
        cbase = jnp.minimum(a_lo + k * CHUNK, I_LAST)
        fetch_wait(it_hbm, cbase)
        cend = jnp.where(k == I_CHUNKS - 1, BIG, cbase + CHUNK)
        ng = match_chunk(na, cbase, cend)

        def grp(j, _):
            off = go[pl.ds(j * 16, 16)]
            pos = gb[pl.ds(j * 16, 16)]
            valid = (j * 16 + e_iota < ng) & (pos >= 0)
            tail = valid & (off >= CHUNK)
            main = valid & (off < CHUNK)
            toff = (off + cbase) - I_TAIL

            pos0 = jnp.zeros((16,), jnp.int32) + pos[0]
            posf = jnp.where(valid, pos, pos0)

            def col(c, _):
                v = plsc.load_gather(
                    chk, [zeros16 + c // 8, zeros16 + c % 8, off], mask=main)
                vt = plsc.load_gather(itail, [toff, zeros16 + c], mask=tail)
                wc = plsc.load_gather(wv, [zeros16 + c])
                v = jnp.where(tail, vt, v) * wc
                v = jnp.where(valid, v, jnp.zeros((16,), jnp.float32) + v[0])
                lane = e_iota * EMB_DIM + c
                plsc.store_scatter(stage, [lane], v)
                plsc.store_scatter(gidx, [lane], posf * EMB_DIM + c)
                return 0
            lax.fori_loop(0, EMB_DIM, col, 0, unroll=1)
            pltpu.async_copy(stage, sp_rows.at[gidx], sem2).wait()
            return 0
        lax.fori_loop(0, (ng + 15) // 16, grp, 0, unroll=1)
        return 0

    lax.fori_loop(0, I_CHUNKS, a_chunk, 0, unroll=1)
    plsc.subcore_barrier()

    b_lo = wid * U_SPAN
    b_hi = jnp.where(wid == 31, BIG, b_lo + U_SPAN)
    nb = prescan(b_lo, b_hi, uidx, lambda t: jnp.full((16,), True))

    def b_chunk(k, ocnt):
        cbase = jnp.minimum(b_lo + k * CHUNK, U_LAST)
        fetch_wait(ut_hbm, cbase)
        cend = jnp.where(k == U_CHUNKS - 1, BIG, cbase + CHUNK)
        ng = match_chunk(nb, cbase, cend)

        def grp(j, ocnt):
            off = go[pl.ds(j * 16, 16)]
            pos = gb[pl.ds(j * 16, 16)]
            valid = (j * 16 + e_iota < ng) & (pos >= 0)
            tail = valid & (off >= CHUNK)
            main = valid & (off < CHUNK)
            toff = (off + cbase) - U_TAIL

            pos0 = jnp.zeros((16,), jnp.int32) + pos[0]
            posf = jnp.where(valid, pos, pos0)

            def bidx(c, _):
                lane = e_iota * EMB_DIM + c
                plsc.store_scatter(gidx, [lane], posf * EMB_DIM + c)
                return 0
            lax.fori_loop(0, EMB_DIM, bidx, 0, unroll=1)
            pltpu.async_copy(sp_rows.at[gidx], stage, sem2).wait()

            def col(c, acc):
                v = plsc.load_gather(
                    chk, [zeros16 + c // 8, zeros16 + c % 8, off], mask=main)
                vt = plsc.load_gather(utail, [toff, zeros16 + c], mask=tail)
                iv = plsc.load_gather(
                    stage, [e_iota * EMB_DIM + c], mask=valid)
                return acc + jnp.where(tail, vt, v) * iv
            acc = lax.fori_loop(0, EMB_DIM, col,
                                jnp.zeros((16,), jnp.float32), unroll=1)
            sig = 1.0 / (1.0 + jnp.exp(-acc))
            sig = jnp.where(valid, sig,
                            jnp.zeros((16,), jnp.float32) + sig[0])
            ovals[pl.ds(ocnt, 16)] = sig
            opos[pl.ds(ocnt, 16)] = posf
            return ocnt + plsc.all_reduce_population_count(valid)[0]
        return lax.fori_loop(0, (ng + 15) // 16, grp, ocnt, unroll=1)

    ocnt = lax.fori_loop(0, U_CHUNKS, b_chunk, jnp.int32(0), unroll=1)

    p0 = jnp.zeros((16,), jnp.int32) + opos[pl.ds(0, 16)][0]
    v0 = jnp.zeros((16,), jnp.float32) + ovals[pl.ds(0, 16)][0]

    def fill(t, _):
        o = jnp.minimum(ocnt + t * 16, CAP)
        ovals[pl.ds(o, 16)] = v0
        opos[pl.ds(o, 16)] = p0
        return 0
    lax.fori_loop(0, CAP // 16 + 1, fill, 0, unroll=1)
    pltpu.async_copy(ovals, out_hbm.at[opos], sem2).wait()


def kernel(user, item, user_table, item_table, W):
    mesh = plsc.VectorSubcoreMesh(core_axis_name="c", subcore_axis_name="s")
    ut3 = user_table.T.reshape(4, 8, N_USERS)
    it3 = item_table.T.reshape(4, 8, N_ITEMS)
    ut_tail = user_table[U_TAIL:]
    it_tail = item_table[I_TAIL:]
    k = pl.kernel(
        _gmf_body,
        out_type=jax.ShapeDtypeStruct((BATCH,), jnp.float32),
        mesh=mesh,
        scratch_types=[
            pltpu.VMEM((BATCH,), jnp.int32),
            pltpu.VMEM((BATCH,), jnp.int32),
            pltpu.VMEM((4, 8, CHUNK), jnp.float32),
            pltpu.VMEM((CAP + 16,), jnp.int32),
            pltpu.VMEM((CAP + 16,), jnp.int32),
            pltpu.VMEM((GCAP + 16,), jnp.int32),
            pltpu.VMEM((GCAP + 16,), jnp.int32),
            pltpu.VMEM((16 * EMB_DIM,), jnp.float32),
            pltpu.VMEM((16 * EMB_DIM,), jnp.int32),
            pltpu.VMEM((N_USERS - U_TAIL, EMB_DIM), jnp.float32),
            pltpu.VMEM((N_ITEMS - I_TAIL, EMB_DIM), jnp.float32),
            pltpu.VMEM((EMB_DIM,), jnp.float32),
            pltpu.VMEM((CAP + 16,), jnp.float32),
            pltpu.VMEM((CAP + 16,), jnp.int32),
            pltpu.VMEM_SHARED((BATCH * EMB_DIM,), jnp.float32),
            pltpu.SemaphoreType.DMA,
            pltpu.SemaphoreType.DMA,
        ],
        compiler_params=pltpu.CompilerParams(
            needs_layout_passes=False, use_tc_tiling_on_sc=True),
    )
    return k(ut3, it3, user.astype(jnp.int32), item.astype(jnp.int32),
             W.reshape(EMB_DIM), ut_tail, it_tail)


# speedup vs baseline: 1.0034x; 1.0034x over previous
import jax
import jax.numpy as jnp
from jax import lax
from jax.experimental import pallas as pl
from jax.experimental.pallas import tpu as pltpu
from jax.experimental.pallas import tpu_sc as plsc

BATCH = 16384
N_USERS = 1000000
N_ITEMS = 100000
EMB_DIM = 32
CHUNK = 1024
U_TAIL = (N_USERS // 128) * 128
I_TAIL = (N_ITEMS // 128) * 128
U_SPAN = 31744
U_CHUNKS = 31
U_LAST = U_TAIL - CHUNK
I_SPAN = 7168
I_CHUNKS = 7
I_LAST = I_TAIL - CHUNK
CAP = 1024
GCAP = 256
BIG = 1 << 28


def _gmf_body(ut_hbm, it_hbm, u_hbm, i_hbm, w_hbm, ut_tl_hbm, it_tl_hbm,
              out_hbm,
              uidx, iidx, chk, lv, lb, go, gb, stage, gidx, utail, itail,
              wv, ovals, opos, sp_rows, sem, sem2):
    nc = lax.axis_index("c")
    ns = lax.axis_index("s")
    wid = ns * 2 + nc
    e_iota = lax.iota(jnp.int32, 16)
    zeros16 = jnp.zeros((16,), jnp.int32)

    pltpu.sync_copy(u_hbm, uidx)
    pltpu.sync_copy(i_hbm, iidx)
    pltpu.sync_copy(w_hbm, wv)
    pltpu.sync_copy(ut_tl_hbm, utail)
    pltpu.sync_copy(it_tl_hbm, itail)

    neg1 = jnp.full((16,), -1, jnp.int32)

    def clear(r, n):
        def bd(t, _):
            r[pl.ds(t * 16, 16)] = neg1
            return 0
        lax.fori_loop(0, n // 16, bd, 0, unroll=1)

    def prescan(lo, hi, vals_ref, extra_filter):
        def bd(t, cnt):
            v = vals_ref[pl.ds(t * 16, 16)]
            b = t * 16 + e_iota
            m = (v >= lo) & (v < hi) & extra_filter(t)
            plsc.store_compressed(lv.at[pl.ds(cnt, 16)], v, mask=m)
            plsc.store_compressed(lb.at[pl.ds(cnt, 16)], b, mask=m)
            return cnt + plsc.all_reduce_population_count(m)[0]
        return lax.fori_loop(0, BATCH // 16, bd, 0, unroll=4)

    def match_chunk(n, cbase, cend):
        def bd(t, cnt):
            v = lv[pl.ds(t * 16, 16)]
            b = lb[pl.ds(t * 16, 16)]
            m = (v >= cbase) & (v < cend) & (t * 16 + e_iota < n)
            plsc.store_compressed(go.at[pl.ds(cnt, 16)], v - cbase, mask=m)
            plsc.store_compressed(gb.at[pl.ds(cnt, 16)], b, mask=m)
            return cnt + plsc.all_reduce_population_count(m)[0]
        return lax.fori_loop(0, (n + 15) // 16, bd, 0)

    def fetch_wait(tbl, cbase):
        cbase = pl.multiple_of(cbase, 128)
        cps = [pltpu.async_copy(tbl.at[fb, :, pl.ds(cbase, CHUNK)],
                                chk.at[fb], sem) for fb in range(4)]
        for c in cps:
            c.wait()

    def sc_half(t):
        u = uidx[pl.ds(t * 16, 16)]
        return ((u // U_SPAN) & 1) == nc

    a_lo = ns * I_SPAN
    na = prescan(a_lo, a_lo + I_SPAN, iidx, sc_half)

    def a_chunk(k, _):
        cbase = jnp.minimum(a_lo + k * CHUNK, I_LAST)
        fetch_wait(it_hbm, cbase)
        cend = jnp.where(k == I_CHUNKS - 1, BIG, cbase + CHUNK)
        ng = match_chunk(na, cbase, cend)

        def grp(j, _):
            off = go[pl.ds(j * 16, 16)]
            pos = gb[pl.ds(j * 16, 16)]
            valid = (j * 16 + e_iota < ng) & (pos >= 0)
            tail = valid & (off >= CHUNK)
            main = valid & (off < CHUNK)
            toff = (off + cbase) - I_TAIL

            pos0 = jnp.zeros((16,), jnp.int32) + pos[0]
            posf = jnp.where(valid, pos, pos0)

            def col(c, _):
                v = plsc.load_gather(
                    chk, [zeros16 + c // 8, zeros16 + c % 8, off], mask=main)
                vt = plsc.load_gather(itail, [toff, zeros16 + c], mask=tail)
                wc = plsc.load_gather(wv, [zeros16 + c])
                v = jnp.where(tail, vt, v) * wc
                v = jnp.where(valid, v, jnp.zeros((16,), jnp.float32) + v[0])
                lane = e_iota * EMB_DIM + c
                plsc.store_scatter(stage, [lane], v)
                plsc.store_scatter(gidx, [lane], posf * EMB_DIM + c)
                return 0
            lax.fori_loop(0, EMB_DIM, col, 0, unroll=8)
            pltpu.async_copy(stage, sp_rows.at[gidx], sem2).wait()
            return 0
        lax.fori_loop(0, (ng + 15) // 16, grp, 0, unroll=1)
        return 0

    lax.fori_loop(0, I_CHUNKS, a_chunk, 0, unroll=1)
    plsc.subcore_barrier()

    b_lo = wid * U_SPAN
    b_hi = jnp.where(wid == 31, BIG, b_lo + U_SPAN)
    nb = prescan(b_lo, b_hi, uidx, lambda t: jnp.full((16,), True))

    def b_chunk(k, ocnt):
        cbase = jnp.minimum(b_lo + k * CHUNK, U_LAST)
        fetch_wait(ut_hbm, cbase)
        cend = jnp.where(k == U_CHUNKS - 1, BIG, cbase + CHUNK)
        ng = match_chunk(nb, cbase, cend)

        def grp(j, ocnt):
            off = go[pl.ds(j * 16, 16)]
            pos = gb[pl.ds(j * 16, 16)]
            valid = (j * 16 + e_iota < ng) & (pos >= 0)
            tail = valid & (off >= CHUNK)
            main = valid & (off < CHUNK)
            toff = (off + cbase) - U_TAIL

            pos0 = jnp.zeros((16,), jnp.int32) + pos[0]
            posf = jnp.where(valid, pos, pos0)

            def bidx(c, _):
                lane = e_iota * EMB_DIM + c
                plsc.store_scatter(gidx, [lane], posf * EMB_DIM + c)
                return 0
            lax.fori_loop(0, EMB_DIM, bidx, 0, unroll=8)
            pltpu.async_copy(sp_rows.at[gidx], stage, sem2).wait()

            def col(c, acc):
                v = plsc.load_gather(
                    chk, [zeros16 + c // 8, zeros16 + c % 8, off], mask=main)
                vt = plsc.load_gather(utail, [toff, zeros16 + c], mask=tail)
                iv = plsc.load_gather(
                    stage, [e_iota * EMB_DIM + c], mask=valid)
                return acc + jnp.where(tail, vt, v) * iv
            acc = lax.fori_loop(0, EMB_DIM, col,
                                jnp.zeros((16,), jnp.float32), unroll=8)
            sig = 1.0 / (1.0 + jnp.exp(-acc))
            sig = jnp.where(valid, sig,
                            jnp.zeros((16,), jnp.float32) + sig[0])
            ovals[pl.ds(ocnt, 16)] = sig
            opos[pl.ds(ocnt, 16)] = posf
            return ocnt + plsc.all_reduce_population_count(valid)[0]
        return lax.fori_loop(0, (ng + 15) // 16, grp, ocnt, unroll=1)

    ocnt = lax.fori_loop(0, U_CHUNKS, b_chunk, jnp.int32(0), unroll=1)

    p0 = jnp.zeros((16,), jnp.int32) + opos[pl.ds(0, 16)][0]
    v0 = jnp.zeros((16,), jnp.float32) + ovals[pl.ds(0, 16)][0]

    def fill(t, _):
        o = jnp.minimum(ocnt + t * 16, CAP)
        ovals[pl.ds(o, 16)] = v0
        opos[pl.ds(o, 16)] = p0
        return 0
    lax.fori_loop(0, CAP // 16 + 1, fill, 0, unroll=1)
    pltpu.async_copy(ovals, out_hbm.at[opos], sem2).wait()


def kernel(user, item, user_table, item_table, W):
    mesh = plsc.VectorSubcoreMesh(core_axis_name="c", subcore_axis_name="s")
    ut3 = user_table.T.reshape(4, 8, N_USERS)
    it3 = item_table.T.reshape(4, 8, N_ITEMS)
    ut_tail = user_table[U_TAIL:]
    it_tail = item_table[I_TAIL:]
    k = pl.kernel(
        _gmf_body,
        out_type=jax.ShapeDtypeStruct((BATCH,), jnp.float32),
        mesh=mesh,
        scratch_types=[
            pltpu.VMEM((BATCH,), jnp.int32),
            pltpu.VMEM((BATCH,), jnp.int32),
            pltpu.VMEM((4, 8, CHUNK), jnp.float32),
            pltpu.VMEM((CAP + 16,), jnp.int32),
            pltpu.VMEM((CAP + 16,), jnp.int32),
            pltpu.VMEM((GCAP + 16,), jnp.int32),
            pltpu.VMEM((GCAP + 16,), jnp.int32),
            pltpu.VMEM((16 * EMB_DIM,), jnp.float32),
            pltpu.VMEM((16 * EMB_DIM,), jnp.int32),
            pltpu.VMEM((N_USERS - U_TAIL, EMB_DIM), jnp.float32),
            pltpu.VMEM((N_ITEMS - I_TAIL, EMB_DIM), jnp.float32),
            pltpu.VMEM((EMB_DIM,), jnp.float32),
            pltpu.VMEM((CAP + 16,), jnp.float32),
            pltpu.VMEM((CAP + 16,), jnp.int32),
            pltpu.VMEM_SHARED((BATCH * EMB_DIM,), jnp.float32),
            pltpu.SemaphoreType.DMA,
            pltpu.SemaphoreType.DMA,
        ],
        compiler_params=pltpu.CompilerParams(
            needs_layout_passes=False, use_tc_tiling_on_sc=True),
    )
    return k(ut3, it3, user.astype(jnp.int32), item.astype(jnp.int32),
             W.reshape(EMB_DIM), ut_tail, it_tail)


# speedup vs baseline: 1.0670x; 1.0634x over previous
import functools

import jax
import jax.numpy as jnp
from jax import lax
from jax.experimental import pallas as pl
from jax.experimental.pallas import tpu as pltpu
from jax.experimental.pallas import tpu_sc as plsc

BATCH = 16384
EMB_DIM = 32
CHUNK = 128


def _gmf_body(b_per_w, n_chunks, user_hbm, item_hbm, utab_hbm, itab_hbm,
              w_hbm, out_hbm, idx_u, idx_i, urows, irows, wv, outv, sem):
    nc = lax.axis_index("c")
    ns = lax.axis_index("s")
    wid = ns * 2 + nc
    crow0 = wid * n_chunks
    base = wid * b_per_w

    pltpu.sync_copy(user_hbm.at[pl.ds(crow0, n_chunks)], idx_u)
    pltpu.sync_copy(item_hbm.at[pl.ds(crow0, n_chunks)], idx_i)
    pltpu.sync_copy(w_hbm, wv)

    copies = []
    for j in range(n_chunks):
        copies.append(pltpu.async_copy(
            utab_hbm.at[idx_u.at[j]], urows.at[pl.ds(j * CHUNK, CHUNK)], sem))
        copies.append(pltpu.async_copy(
            itab_hbm.at[idx_i.at[j]], irows.at[pl.ds(j * CHUNK, CHUNK)], sem))
    for c in copies:
        c.wait()

    w_lo = wv[pl.ds(0, 16)]
    w_hi = wv[pl.ds(16, 16)]
    w_s = [w_lo[c] for c in range(16)] + [w_hi[c] for c in range(16)]

    def group(g, _):
        rows = g * 16 + lax.iota(jnp.int32, 16)
        acc = jnp.zeros((16,), jnp.float32)
        for c in range(EMB_DIM):
            col = jnp.full((16,), c, jnp.int32)
            u = plsc.load_gather(urows, [rows, col])
            it = plsc.load_gather(irows, [rows, col])
            acc = acc + (u * it) * w_s[c]
        outv[pl.ds(g * 16, 16)] = 1.0 / (1.0 + jnp.exp(-acc))
        return 0

    lax.fori_loop(0, b_per_w // 16, group, 0)
    pltpu.sync_copy(outv, out_hbm.at[pl.ds(base, b_per_w)])


def kernel(user, item, user_table, item_table, W):
    info = plsc.get_sparse_core_info()
    nw = info.num_cores * info.num_subcores
    assert info.num_cores == 2 and info.num_lanes == 16
    b_per_w = BATCH // nw
    n_chunks = b_per_w // CHUNK

    mesh = plsc.VectorSubcoreMesh(core_axis_name="c", subcore_axis_name="s")
    grid_kernel = pl.kernel(
        functools.partial(_gmf_body, b_per_w, n_chunks),
        out_type=jax.ShapeDtypeStruct((BATCH,), jnp.float32),
        mesh=mesh,
        scratch_types=[
            pltpu.VMEM((n_chunks, CHUNK), jnp.int32),
            pltpu.VMEM((n_chunks, CHUNK), jnp.int32),
            pltpu.VMEM((b_per_w, EMB_DIM), jnp.float32),
            pltpu.VMEM((b_per_w, EMB_DIM), jnp.float32),
            pltpu.VMEM((EMB_DIM,), jnp.float32),
            pltpu.VMEM((b_per_w,), jnp.float32),
            pltpu.SemaphoreType.DMA,
        ],
        compiler_params=pltpu.CompilerParams(
            needs_layout_passes=False, use_tc_tiling_on_sc=False),
    )
    user2d = user.astype(jnp.int32).reshape(nw * n_chunks, CHUNK)
    item2d = item.astype(jnp.int32).reshape(nw * n_chunks, CHUNK)
    return grid_kernel(user2d, item2d, user_table, item_table,
                       W.reshape(EMB_DIM))
